# baseline (speedup 1.0000x reference)
import functools

import jax
import jax.numpy as jnp
from jax import lax
from jax.experimental import pallas as pl
from jax.experimental.pallas import tpu as pltpu
from jax.experimental.pallas import tpu_sc as plsc

N = 10000
D = 128
G = 64
CH = 128
NSUB = 16
NPAD = N + 16
ZROWS = NPAD // NSUB
OROWS = 624
OLAST = N - (NSUB - 1) * OROWS


def _segsum_pair(table, src, dst, cpw):
    mesh = plsc.VectorSubcoreMesh(core_axis_name="c", subcore_axis_name="s")

    RING = 2
    IRING = 4

    @functools.partial(
        pl.kernel,
        out_type=jax.ShapeDtypeStruct((2, N, D), jnp.float32),
        mesh=mesh,
        scratch_types=(
            [pltpu.VMEM((CH,), jnp.int32) for _ in range(IRING)]
            + [pltpu.VMEM((CH,), jnp.int32) for _ in range(IRING)]
            + [pltpu.VMEM((CH, D), jnp.float32) for _ in range(RING)]
            + [pltpu.VMEM((CH, D), jnp.float32)]
            + [pltpu.VMEM_SHARED((NPAD, D), jnp.float32)]
            + [pltpu.SemaphoreType.DMA] * (2 * IRING + 2 * RING)
        ),
    )
    def k(table_hbm, src_hbm, dst_hbm, out_hbm, *scr):
        sidx = scr[0:IRING]
        didx = scr[IRING:2 * IRING]
        rows = scr[2 * IRING:2 * IRING + RING]
        zbuf = scr[2 * IRING + RING]
        acc = scr[2 * IRING + RING + 1]
        sems = scr[2 * IRING + RING + 2:]
        ssem = sems[0:IRING]
        dsem = sems[IRING:2 * IRING]
        gsem = sems[2 * IRING:2 * IRING + RING]
        csem = sems[2 * IRING + RING:]
        c = lax.axis_index("c")
        s = lax.axis_index("s")

        def start_idx(j, i):
            pltpu.async_copy(src_hbm.at[c, s, j], sidx[i], ssem[i])
            pltpu.async_copy(dst_hbm.at[c, s, j], didx[i], dsem[i])

        def wait_idx_s(i):
            pltpu.make_async_copy(src_hbm.at[c, s, 0], sidx[i], ssem[i]).wait()

        def wait_idx_d(i):
            pltpu.make_async_copy(dst_hbm.at[c, s, 0], didx[i], dsem[i]).wait()

        def start_gather(b, i):
            pltpu.async_copy(table_hbm.at[sidx[i]], rows[b], gsem[b])

        def wait_gather(b):
            pltpu.make_async_copy(table_hbm.at[pl.ds(0, CH)], rows[b],
                                  gsem[b]).wait()

        def start_scatter(b, i):
            pltpu.async_copy(rows[b], acc.at[didx[i]], csem[b], add=True)

        def wait_scatter(b):
            pltpu.make_async_copy(table_hbm.at[pl.ds(0, CH)], rows[b],
                                  csem[b]).wait()

        start_idx(0, 0)
        start_idx(1, 1)
        start_idx(2, 2)
        wait_idx_s(0)
        start_gather(0, 0)
        wait_idx_s(1)
        start_gather(1, 1)

        @pl.loop(0, CH)
        def _(r):
            @pl.loop(0, D, step=16)
            def _(col):
                zbuf.at[pl.ds(r, 1), pl.ds(col, 16)][...] = jnp.zeros(
                    (1, 16), jnp.float32)

        base = s * ZROWS
        pltpu.sync_copy(zbuf.at[pl.ds(0, CH)], acc.at[pl.ds(base, CH)])
        pltpu.sync_copy(zbuf.at[pl.ds(0, CH)], acc.at[pl.ds(base + CH, CH)])
        pltpu.sync_copy(zbuf.at[pl.ds(0, CH)],
                        acc.at[pl.ds(base + 2 * CH, CH)])
        pltpu.sync_copy(zbuf.at[pl.ds(0, CH)],
                        acc.at[pl.ds(base + 3 * CH, CH)])
        pltpu.sync_copy(zbuf.at[pl.ds(0, ZROWS - 4 * CH)],
                        acc.at[pl.ds(base + 4 * CH, ZROWS - 4 * CH)])
        plsc.subcore_barrier()

        @pl.loop(0, cpw - 4, step=4)
        def _(jj):
            for t in range(4):
                b = t % 2
                wait_gather(b)
                wait_idx_d(t)
                pltpu.sync_copy(rows[b], acc.at[didx[t]], add=True)
                start_idx(jj + t + 3, (t + 3) % IRING)
                wait_idx_s((t + 2) % IRING)
                start_gather(b, (t + 2) % IRING)

        for j in range(cpw - 4, cpw):
            t = j % IRING
            b = j % 2
            wait_gather(b)
            wait_idx_d(t)
            pltpu.sync_copy(rows[b], acc.at[didx[t]], add=True)
            if j + 3 < cpw:
                start_idx(j + 3, (j + 3) % IRING)
            if j + 2 < cpw:
                wait_idx_s((j + 2) % IRING)
                start_gather(b, (j + 2) % IRING)
        plsc.subcore_barrier()
        ob = s * OROWS

        @pl.when(s < NSUB - 1)
        def _():
            pltpu.sync_copy(acc.at[pl.ds(ob, OROWS)],
                            out_hbm.at[c, pl.ds(ob, OROWS)])

        @pl.when(s == NSUB - 1)
        def _():
            pltpu.sync_copy(acc.at[pl.ds((NSUB - 1) * OROWS, OLAST)],
                            out_hbm.at[c, pl.ds((NSUB - 1) * OROWS, OLAST)])

    return k(table, src, dst)


def _dot_t(a, w):
    return lax.dot_general(a, w, (((1,), (1,)), ((), ())),
                           preferred_element_type=jnp.float32)


def _tc_layer(x, s0, s1, Wa, ba, Wb, bb):
    def body(x_ref, s0_ref, s1_ref, wa_ref, ba_ref, wb_ref, bb_ref, o_ref):
        m0 = _dot_t(x_ref[...] + s0_ref[...], wa_ref[...])
        m1 = _dot_t(x_ref[...] + s1_ref[...], wb_ref[...])
        o_ref[...] = jnp.maximum(m0 + ba_ref[...] + m1 + bb_ref[...], 0.0)

    return pl.pallas_call(
        body,
        out_shape=jax.ShapeDtypeStruct((N, D), jnp.float32),
    )(x, s0, s1, Wa, ba.reshape(1, D), Wb, bb.reshape(1, D))


def _tc_final(h, s0, s1, Wa, ba, Wb, bb, batch2d, Wh0, bh0, Wh1, bh1, out_dim):
    def body(h_ref, s0_ref, s1_ref, wa_ref, ba_ref, wb_ref, bb_ref, b_ref,
             wh0_ref, bh0_ref, wh1_ref, bh1_ref, o_ref):
        m0 = _dot_t(h_ref[...] + s0_ref[...], wa_ref[...])
        m1 = _dot_t(h_ref[...] + s1_ref[...], wb_ref[...])
        h2 = jnp.maximum(m0 + ba_ref[...] + m1 + bb_ref[...], 0.0)
        gids = lax.broadcasted_iota(jnp.int32, (G, N), 0)
        onehot = jnp.where(gids == b_ref[...], 1.0, 0.0)
        sums = jnp.dot(onehot, h2, preferred_element_type=jnp.float32)
        counts = jnp.sum(onehot, axis=1, keepdims=True)
        pooled = sums / jnp.maximum(counts, 1.0)
        z = jnp.maximum(_dot_t(pooled, wh0_ref[...]) + bh0_ref[...], 0.0)
        o_ref[...] = _dot_t(z, wh1_ref[...]) + bh1_ref[...]

    return pl.pallas_call(
        body,
        out_shape=jax.ShapeDtypeStruct((G, out_dim), jnp.float32),
    )(h, s0, s1, Wa, ba.reshape(1, D), Wb, bb.reshape(1, D), batch2d,
      Wh0, bh0.reshape(1, D), Wh1, bh1.reshape(1, out_dim))


def kernel(x, edge_index_e0, edge_index_e1, batch,
           W0_e0, b0_e0, W0_e1, b0_e1,
           W1_e0, b1_e0, W1_e1, b1_e1,
           Wh0, bh0, Wh1, bh1):
    e = edge_index_e0.shape[1]
    cpw = -(-e // (NSUB * CH))
    cpw = -(-cpw // 4) * 4
    epad = NSUB * CH * cpw
    pad = epad - e

    def prep(ei):
        src = jnp.concatenate([ei[0], jnp.zeros((pad,), jnp.int32)])
        dst = jnp.concatenate([ei[1], jnp.full((pad,), N, jnp.int32)])
        return src, dst

    s0_, d0_ = prep(edge_index_e0)
    s1_, d1_ = prep(edge_index_e1)
    src = jnp.stack([s0_, s1_]).reshape(2, NSUB, cpw, CH)
    dst = jnp.stack([d0_, d1_]).reshape(2, NSUB, cpw, CH)

    agg0 = _segsum_pair(x, src, dst, cpw)
    h1 = _tc_layer(x, agg0[0], agg0[1], W0_e0, b0_e0, W0_e1, b0_e1)
    agg1 = _segsum_pair(h1, src, dst, cpw)
    out = _tc_final(h1, agg1[0], agg1[1], W1_e0, b1_e0, W1_e1, b1_e1,
                    batch.reshape(1, N), Wh0, bh0, Wh1, bh1, Wh1.shape[0])
    return out


# speedup vs baseline: 1.5212x; 1.5212x over previous
import functools

import jax
import jax.numpy as jnp
from jax import lax
from jax.experimental import pallas as pl
from jax.experimental.pallas import tpu as pltpu
from jax.experimental.pallas import tpu_sc as plsc

N = 10000
D = 128
G = 64
CH = 128
NSUB = 16
NPAD = N + 16
ZROWS = NPAD // NSUB
OROWS = 624
OLAST = N - (NSUB - 1) * OROWS


def _segsum_pair(table, eidx, cpw):
    mesh = plsc.VectorSubcoreMesh(core_axis_name="c", subcore_axis_name="s")

    RING = 2

    @functools.partial(
        pl.kernel,
        out_type=jax.ShapeDtypeStruct((2, N, D), jnp.float32),
        mesh=mesh,
        scratch_types=(
            [pltpu.VMEM((2, CH), jnp.int32) for _ in range(RING)]
            + [pltpu.VMEM((CH, D), jnp.float32) for _ in range(RING)]
            + [pltpu.VMEM((CH, D), jnp.float32)]
            + [pltpu.VMEM_SHARED((NPAD, D), jnp.float32)]
            + [pltpu.SemaphoreType.DMA] * (2 * RING)
        ),
    )
    def k(table_hbm, eidx_hbm, out_hbm, *scr):
        idx = scr[0:RING]
        rows = scr[RING:2 * RING]
        zbuf = scr[2 * RING]
        acc = scr[2 * RING + 1]
        sems = scr[2 * RING + 2:]
        isem = sems[0:RING]
        gsem = sems[RING:2 * RING]
        c = lax.axis_index("c")
        s = lax.axis_index("s")

        def start_idx(j, i):
            pltpu.async_copy(eidx_hbm.at[c, s, j], idx[i], isem[i])

        def wait_idx(i):
            pltpu.make_async_copy(eidx_hbm.at[c, s, 0], idx[i], isem[i]).wait()

        def start_gather(b, i):
            pltpu.async_copy(table_hbm.at[idx[i].at[0]], rows[b], gsem[b])

        def wait_gather(b):
            pltpu.make_async_copy(table_hbm.at[pl.ds(0, CH)], rows[b],
                                  gsem[b]).wait()

        start_idx(0, 0)
        start_idx(1, 1)

        @pl.loop(0, CH)
        def _(r):
            @pl.loop(0, D, step=16)
            def _(col):
                zbuf.at[pl.ds(r, 1), pl.ds(col, 16)][...] = jnp.zeros(
                    (1, 16), jnp.float32)

        base = s * ZROWS
        pltpu.sync_copy(zbuf.at[pl.ds(0, CH)], acc.at[pl.ds(base, CH)])
        pltpu.sync_copy(zbuf.at[pl.ds(0, CH)], acc.at[pl.ds(base + CH, CH)])
        pltpu.sync_copy(zbuf.at[pl.ds(0, CH)],
                        acc.at[pl.ds(base + 2 * CH, CH)])
        pltpu.sync_copy(zbuf.at[pl.ds(0, CH)],
                        acc.at[pl.ds(base + 3 * CH, CH)])
        pltpu.sync_copy(zbuf.at[pl.ds(0, ZROWS - 4 * CH)],
                        acc.at[pl.ds(base + 4 * CH, ZROWS - 4 * CH)])
        plsc.subcore_barrier()

        wait_idx(0)
        start_gather(0, 0)

        @pl.loop(0, cpw - 2, step=2)
        def _(jj):
            for b in (0, 1):
                j = jj + b
                wait_idx(1 - b)
                start_gather(1 - b, 1 - b)
                wait_gather(b)
                pltpu.sync_copy(rows[b], acc.at[idx[b].at[1]], add=True)
                start_idx(j + 2, b)

        wait_idx(1)
        start_gather(1, 1)
        wait_gather(0)
        pltpu.sync_copy(rows[0], acc.at[idx[0].at[1]], add=True)
        wait_gather(1)
        pltpu.sync_copy(rows[1], acc.at[idx[1].at[1]], add=True)
        plsc.subcore_barrier()
        ob = s * OROWS

        @pl.when(s < NSUB - 1)
        def _():
            pltpu.sync_copy(acc.at[pl.ds(ob, OROWS)],
                            out_hbm.at[c, pl.ds(ob, OROWS)])

        @pl.when(s == NSUB - 1)
        def _():
            pltpu.sync_copy(acc.at[pl.ds((NSUB - 1) * OROWS, OLAST)],
                            out_hbm.at[c, pl.ds((NSUB - 1) * OROWS, OLAST)])

    return k(table, eidx)


def _dot_t(a, w):
    return lax.dot_general(a, w, (((1,), (1,)), ((), ())),
                           preferred_element_type=jnp.float32)


def _tc_layer(x, s0, s1, Wa, ba, Wb, bb):
    def body(x_ref, s0_ref, s1_ref, wa_ref, ba_ref, wb_ref, bb_ref, o_ref):
        m0 = _dot_t(x_ref[...] + s0_ref[...], wa_ref[...])
        m1 = _dot_t(x_ref[...] + s1_ref[...], wb_ref[...])
        o_ref[...] = jnp.maximum(m0 + ba_ref[...] + m1 + bb_ref[...], 0.0)

    return pl.pallas_call(
        body,
        out_shape=jax.ShapeDtypeStruct((N, D), jnp.float32),
    )(x, s0, s1, Wa, ba.reshape(1, D), Wb, bb.reshape(1, D))


def _tc_final(h, s0, s1, Wa, ba, Wb, bb, batch2d, Wh0, bh0, Wh1, bh1, out_dim):
    def body(h_ref, s0_ref, s1_ref, wa_ref, ba_ref, wb_ref, bb_ref, b_ref,
             wh0_ref, bh0_ref, wh1_ref, bh1_ref, o_ref):
        m0 = _dot_t(h_ref[...] + s0_ref[...], wa_ref[...])
        m1 = _dot_t(h_ref[...] + s1_ref[...], wb_ref[...])
        h2 = jnp.maximum(m0 + ba_ref[...] + m1 + bb_ref[...], 0.0)
        gids = lax.broadcasted_iota(jnp.int32, (G, N), 0)
        onehot = jnp.where(gids == b_ref[...], 1.0, 0.0)
        sums = jnp.dot(onehot, h2, preferred_element_type=jnp.float32)
        counts = jnp.sum(onehot, axis=1, keepdims=True)
        pooled = sums / jnp.maximum(counts, 1.0)
        z = jnp.maximum(_dot_t(pooled, wh0_ref[...]) + bh0_ref[...], 0.0)
        o_ref[...] = _dot_t(z, wh1_ref[...]) + bh1_ref[...]

    return pl.pallas_call(
        body,
        out_shape=jax.ShapeDtypeStruct((G, out_dim), jnp.float32),
    )(h, s0, s1, Wa, ba.reshape(1, D), Wb, bb.reshape(1, D), batch2d,
      Wh0, bh0.reshape(1, D), Wh1, bh1.reshape(1, out_dim))


def kernel(x, edge_index_e0, edge_index_e1, batch,
           W0_e0, b0_e0, W0_e1, b0_e1,
           W1_e0, b1_e0, W1_e1, b1_e1,
           Wh0, bh0, Wh1, bh1):
    e = edge_index_e0.shape[1]
    cpw = -(-e // (NSUB * CH))
    cpw += cpw % 2
    epad = NSUB * CH * cpw
    pad = epad - e

    def prep(ei):
        src = jnp.concatenate([ei[0], jnp.zeros((pad,), jnp.int32)])
        dst = jnp.concatenate([ei[1], jnp.full((pad,), N, jnp.int32)])
        return src, dst

    s0_, d0_ = prep(edge_index_e0)
    s1_, d1_ = prep(edge_index_e1)
    src = jnp.stack([s0_, s1_]).reshape(2, NSUB, cpw, CH)
    dst = jnp.stack([d0_, d1_]).reshape(2, NSUB, cpw, CH)
    eidx = jnp.stack([src, dst], axis=3)

    agg0 = _segsum_pair(x, eidx, cpw)
    h1 = _tc_layer(x, agg0[0], agg0[1], W0_e0, b0_e0, W0_e1, b0_e1)
    agg1 = _segsum_pair(h1, eidx, cpw)
    out = _tc_final(h1, agg1[0], agg1[1], W1_e0, b1_e0, W1_e1, b1_e1,
                    batch.reshape(1, N), Wh0, bh0, Wh1, bh1, Wh1.shape[0])
    return out
